# Initial kernel scaffold; baseline (speedup 1.0000x reference)
import jax
import jax.numpy as jnp
from jax.experimental import pallas as pl


def kernel(adj, node_feat, edge_feat, params):
    raise NotImplementedError("write your pallas kernel here")


# speedup vs baseline: 7.1413x; 7.1413x over previous
import jax
import jax.numpy as jnp
from jax import lax
from jax.experimental import pallas as pl
from jax.experimental.pallas import tpu as pltpu

F32 = jnp.float32
I32 = jnp.int32
N0 = 1024
E0 = 16384
C0 = 128
C1 = 32
_INTERPRET = False
_PREC = lax.Precision.DEFAULT


def _run(body, out_specs, scratch_shapes, *args):
    flat, tree = jax.tree.flatten(args)
    n_in = len(flat)
    n_scr = len(scratch_shapes)

    def wrapped(*refs):
        ins = jax.tree.unflatten(tree, refs[:n_in])
        outs = refs[n_in:len(refs) - n_scr] if n_scr else refs[n_in:]
        scr = refs[len(refs) - n_scr:] if n_scr else ()
        body(ins, outs, scr)

    return pl.pallas_call(
        wrapped,
        out_shape=[jax.ShapeDtypeStruct(s, d) for s, d in out_specs],
        scratch_shapes=list(scratch_shapes),
        interpret=_INTERPRET,
    )(*flat)


def _lin(p, x):
    return jnp.dot(x, p["w"].T, preferred_element_type=F32, precision=_PREC) + p["b"]


def _leaky(x):
    return jnp.maximum(x, 0.2 * x)


def _sigmoid(x):
    return 1.0 / (1.0 + jnp.exp(-x))


BF = jnp.bfloat16


def _onehot_t(idx_row, n):
    return (lax.broadcasted_iota(I32, (n, idx_row.shape[1]), 0)
            == idx_row).astype(BF)


def _ddot(a, b):
    return lax.dot_general(a, b, (((0,), (0,)), ((), ())),
                           preferred_element_type=F32, precision=_PREC)


def _hilo_bf(x):
    hi = x.astype(BF)
    return hi, (x - hi.astype(F32)).astype(BF)


def _ddot_x(oh_bf, x):
    hi, lo = _hilo_bf(x)
    dn = (((0,), (0,)), ((), ()))
    return (lax.dot_general(oh_bf, hi, dn, preferred_element_type=F32)
            + lax.dot_general(oh_bf, lo, dn, preferred_element_type=F32))


def _dot_x(oh_bf, x):
    hi, lo = _hilo_bf(x)
    return (jnp.dot(oh_bf, hi, preferred_element_type=F32)
            + jnp.dot(oh_bf, lo, preferred_element_type=F32))


def _argmax_col(z, ncol):
    rowmax = jnp.max(z, axis=1, keepdims=True)
    io = lax.broadcasted_iota(I32, z.shape, 1)
    return jnp.min(jnp.where(z == rowmax, io, ncol), axis=1, keepdims=True)


_EB = 1024
_EBP = 512


def _level0_body(ins, outs, scr):
    src_r, dst_r, nf_r, ef_r, g0_r, P_r = ins
    blat_o, shrlat_o, c0_o, eh_o, lat0_o = outs
    ae_s, = scr
    P = jax.tree.map(lambda r: r[...], P_r)

    iota_ne = lax.broadcasted_iota(I32, (N0, _EB), 0)

    def edge_rows(i, eb):
        off = i * eb
        s_row = jnp.reshape(src_r[pl.ds(off, eb)], (1, eb))
        d_row = jnp.reshape(dst_r[pl.ds(off, eb)], (1, eb))
        return off, (iota_ne == s_row).astype(BF), (iota_ne == d_row).astype(BF)

    def gat_layer(gp, h_in, lidx, fill, ae_max, count_col):
        hh = jnp.dot(h_in, gp["lin"].T, preferred_element_type=F32, precision=_PREC)
        a_s = jnp.sum(hh * gp["att_src"], axis=1, keepdims=True)
        a_d = jnp.sum(hh * gp["att_dst"], axis=1, keepdims=True)
        e_loop = jnp.dot(fill, gp["lin_edge"].T,
                         preferred_element_type=F32, precision=_PREC)
        a_loop = jnp.sum(e_loop * gp["att_edge"], axis=1,
                         keepdims=True)
        M = _leaky(jnp.max(a_s) + jnp.max(a_d)
                   + jnp.maximum(ae_max, a_loop[0, 0]))
        table = jnp.concatenate([hh, a_s], axis=1)
        width = 34 if count_col else 33

        def body(i, acc):
            off, ohs_t, ohd_t = edge_rows(i, _EB)
            g = _ddot_x(ohs_t, table)
            ad = _ddot_x(ohd_t, a_d)
            ae = ae_s[pl.ds(off, _EB), lidx:lidx + 1]
            w = jnp.exp(_leaky(g[:, 32:33] + ad + ae) - M)
            cols = [w * g[:, :32], w]
            if count_col:
                cols.append(jnp.ones_like(w))
            return acc + _dot_x(ohd_t, jnp.concatenate(cols, axis=1))

        acc = lax.fori_loop(0, E0 // _EB, body, jnp.zeros((N0, width), F32))
        w_self = jnp.exp(_leaky(a_s + a_d + a_loop) - M)
        out = ((acc[:, :32] + w_self * hh)
               / (acc[:, 32:33] + w_self + 1e-16) + gp["bias"])
        indeg = acc[:, 33:34] if count_col else None
        return out, indeg

    def gcn_aggregate(table, width):
        def body(i, acc):
            _, ohs_t, ohd_t = edge_rows(i, _EB)
            g = _ddot_x(ohs_t, table)
            dis_d = _ddot_x(ohd_t, table[:, width:width + 1])
            w = g[:, width:width + 1] * dis_d
            return acc + _dot_x(ohd_t, w * g[:, :width])

        return lax.fori_loop(0, E0 // _EB, body,
                             jnp.zeros((N0, width), F32))

    be = P["base_enc"]
    def edge_mlp_block(i, carry):
        sum_eh, mx = carry
        off = i * _EB
        ef_blk = ef_r[pl.ds(off, _EB), :]
        t = jnp.tanh(_lin(be["edge_fc1"], ef_blk))
        eh_blk = jnp.tanh(_lin(be["edge_fc2"], t))
        eh_o[pl.ds(off, _EB), :] = eh_blk
        ae_blk = jnp.concatenate(
            [jnp.sum(jnp.dot(eh_blk, gp["lin_edge"].T,
                             preferred_element_type=F32, precision=_PREC)
                     * gp["att_edge"], axis=1, keepdims=True)
             for gp in be["gat"]], axis=1)
        ae_s[pl.ds(off, _EB), :] = ae_blk
        return (sum_eh + jnp.sum(eh_blk, axis=0, keepdims=True),
                jnp.maximum(mx, jnp.max(ae_blk, axis=0, keepdims=True)))

    sum_eh, ae_mx = lax.fori_loop(
        0, E0 // _EB, edge_mlp_block,
        (jnp.zeros((1, 32), F32), jnp.full((1, 2), -1e30, F32)))
    fill = sum_eh / F32(E0)

    node_feat = nf_r[...]
    nh = jnp.tanh(_lin(be["node_fc1"], node_feat))
    nh = jnp.tanh(_lin(be["node_fc2"], nh))
    h1, indeg = gat_layer(be["gat"][0], nh, 0, fill, ae_mx[0, 0], True)
    h2, _ = gat_layer(be["gat"][1], h1, 1, fill, ae_mx[0, 1], False)
    hcat = jnp.concatenate([nh, h1, h2], axis=1)
    lat = jnp.tanh(_lin(be["latent_fc1"], hcat))
    base_lat = jnp.tanh(_lin(be["latent_fc2"], lat))

    dis = 1.0 / jnp.sqrt(indeg + 1.0)
    cp = P["cluster"][0]
    xc = _sigmoid(_lin(cp["fc1"], base_lat))
    xc = _sigmoid(_lin(cp["fc2"], xc))
    h = xc
    for gp in list(cp["gcn"]) + [cp["assign"]]:
        hw = jnp.dot(h, gp["w"].T, preferred_element_type=F32,
                     precision=_PREC)
        width = hw.shape[1]
        agg = gcn_aggregate(jnp.concatenate([hw, dis], axis=1), width)
        h = agg + dis * dis * hw + gp["b"]
    score = h

    c0_col = _argmax_col(score + g0_r[...], C0)
    s0 = (lax.broadcasted_iota(I32, (N0, C0), 1) == c0_col).astype(F32)
    shr_raw = _ddot(s0, base_lat)
    nrm = jnp.sqrt(jnp.sum(shr_raw * shr_raw, axis=0, keepdims=True))
    blat_o[...] = base_lat
    shrlat_o[...] = shr_raw / jnp.maximum(nrm, 1e-12)
    c0_o[...] = jnp.reshape(c0_col, (N0,))
    lat0_o[...] = jnp.sum(base_lat, axis=0, keepdims=True)


def _pool0_body(ins, outs, scr):
    src_r, dst_r, c0_r, eh_r = ins
    adj_o, mask_o = outs
    c0row = jnp.reshape(c0_r[...], (1, N0)).astype(BF)
    iota_n = lax.broadcasted_iota(I32, (N0, _EBP), 0)
    iota_e = lax.broadcasted_iota(I32, (_EBP, C0), 1)
    kcol = C0 * 32
    rexp = (lax.broadcasted_iota(I32, (32, kcol), 1) // C0
            == lax.broadcasted_iota(I32, (32, kcol), 0)).astype(BF)
    texp = (lax.broadcasted_iota(I32, (C0, kcol), 1) % C0
            == lax.broadcasted_iota(I32, (C0, kcol), 0)).astype(BF)

    def body(i, acc):
        off = i * _EBP
        s_row = jnp.reshape(src_r[pl.ds(off, _EBP)], (1, _EBP))
        d_row = jnp.reshape(dst_r[pl.ds(off, _EBP)], (1, _EBP))
        ohs_t = (iota_n == s_row).astype(BF)
        ohd_t = (iota_n == d_row).astype(BF)
        cs = jnp.dot(c0row, ohs_t, preferred_element_type=F32)
        cd = jnp.dot(c0row, ohd_t, preferred_element_type=F32)
        ohcs_t = (lax.broadcasted_iota(I32, (C0, _EBP), 0)
                  == cs.astype(I32)).astype(BF)
        cd_col = jnp.reshape(cd, (_EBP, 1)).astype(I32)
        ohcd = (iota_e == cd_col).astype(BF)
        ph, plo = _hilo_bf(eh_r[pl.ds(off, _EBP), :])
        ohcd_rep = jnp.dot(ohcd, texp, preferred_element_type=F32).astype(BF)
        zh = ohcd_rep * jnp.dot(ph, rexp,
                                preferred_element_type=F32).astype(BF)
        zl = ohcd_rep * jnp.dot(plo, rexp,
                                preferred_element_type=F32).astype(BF)
        return (acc
                + jnp.dot(ohcs_t, zh, preferred_element_type=F32)
                + jnp.dot(ohcs_t, zl, preferred_element_type=F32))

    acc = lax.fori_loop(0, E0 // _EBP, body, jnp.zeros((C0, kcol), F32))
    adj_n = acc / jnp.sum(acc)
    bd2 = (lax.broadcasted_iota(I32, (kcol, C0), 0) % C0
           == lax.broadcasted_iota(I32, (kcol, C0), 1)).astype(BF)
    hi, lo = _hilo_bf(adj_n)
    rs = (jnp.dot(hi, bd2, preferred_element_type=F32)
          + jnp.dot(lo, bd2, preferred_element_type=F32))
    mask_o[...] = (rs != 0).astype(F32)
    adj_o[...] = adj_n


def _vfold(gp):
    return jnp.dot(gp["lin_edge"].T, gp["att_edge"][:, None],
                   preferred_element_type=F32, precision=_PREC)


def _gat_dense(gp, x, ae2d, fill, mask2d, n):
    h = jnp.dot(x, gp["lin"].T, preferred_element_type=F32, precision=_PREC)
    a_s = jnp.sum(h * gp["att_src"], axis=1, keepdims=True)
    a_d = jnp.sum(h * gp["att_dst"], axis=1, keepdims=True)
    e_loop = jnp.dot(fill, gp["lin_edge"].T, preferred_element_type=F32,
                     precision=_PREC)
    a_loop = jnp.sum(e_loop * gp["att_edge"], axis=1, keepdims=True)
    A = _leaky(a_s + jnp.reshape(a_d, (1, n)) + ae2d)
    a_self = _leaky(a_s + a_d + a_loop)
    Am = jnp.where(mask2d > 0, A, -1e30)
    m = jnp.maximum(jnp.max(Am, axis=0, keepdims=True),
                    jnp.reshape(a_self, (1, n)))
    W = jnp.where(mask2d > 0, jnp.exp(A - m), 0.0)
    w_self = jnp.exp(a_self - jnp.reshape(m, (n, 1)))
    den = jnp.reshape(jnp.sum(W, axis=0, keepdims=True), (n, 1)) + w_self
    num = lax.dot_general(W, h, (((0,), (0,)), ((), ())),
                          preferred_element_type=F32,
                          precision=lax.Precision.HIGHEST) + w_self * h
    return num / (den + 1e-16) + gp["bias"]


def _gcn_dense(gp, x, mask2d, n):
    h = jnp.dot(x, gp["w"].T, preferred_element_type=F32, precision=_PREC)
    deg = jnp.sum(mask2d, axis=0, keepdims=True) + 1.0
    dis = 1.0 / jnp.sqrt(deg)
    Nw = jnp.reshape(dis, (n, 1)) * dis * mask2d
    agg = lax.dot_general(Nw, h, (((0,), (0,)), ((), ())),
                          preferred_element_type=F32,
                          precision=lax.Precision.HIGHEST)
    return agg + jnp.reshape(dis * dis, (n, 1)) * h + gp["b"]


def _encoder_dense(p, x, ef, mask2d, mask_col, n):
    nh = jnp.tanh(_lin(p["node_fc1"], x))
    nh = jnp.tanh(_lin(p["node_fc2"], nh))
    eh = jnp.tanh(_lin(p["edge_fc1"], ef))
    eh = jnp.tanh(_lin(p["edge_fc2"], eh))
    fill = jnp.sum(eh * mask_col, axis=0, keepdims=True) / jnp.sum(mask_col)
    h = nh
    hs = [nh]
    for gp in p["gat"]:
        e_edge = jnp.dot(eh, gp["lin_edge"].T, preferred_element_type=F32,
                         precision=_PREC)
        ae2d = jnp.reshape(jnp.sum(e_edge * gp["att_edge"], axis=1,
                                   keepdims=True), (n, n))
        h = _gat_dense(gp, h, ae2d, fill, mask2d, n)
        hs.append(h)
    hcat = jnp.concatenate(hs, axis=1)
    lat = jnp.tanh(_lin(p["latent_fc1"], hcat))
    return jnp.tanh(_lin(p["latent_fc2"], lat)), eh


def _cluster_dense(p, x, mask2d, n):
    h = _sigmoid(_lin(p["fc1"], x))
    h = _sigmoid(_lin(p["fc2"], h))
    for gp in list(p["gcn"]) + [p["assign"]]:
        h = _gcn_dense(gp, h, mask2d, n)
    return h


_EB1 = 2048


def _rest_a_body(ins, outs, scr):
    shrlat_r, sheh0_r, m02d_r, m0col_r, ii_r, jj_r, g1_r, P_r = ins
    sheh1_o, mask1_o, shrlat1_o, nl1_o = outs
    ehm_s, = scr
    P = jax.tree.map(lambda r: r[...], P_r)
    mask0_2d = m02d_r[...]
    gl = P["glob"][0]

    def edge_blk(i, sum_ehm):
        off = i * _EB1
        m_blk = m0col_r[pl.ds(off, _EB1), :]
        ef_blk = sheh0_r[pl.ds(off, _EB1), :] * m_blk
        t = jnp.tanh(_lin(gl["edge_fc1"], ef_blk))
        eh_blk = jnp.tanh(_lin(gl["edge_fc2"], t))
        ehm_s[pl.ds(off, _EB1), 0:32] = eh_blk * m_blk
        ehm_s[pl.ds(off, _EB1), 32:34] = jnp.concatenate(
            [jnp.sum(jnp.dot(eh_blk, gp["lin_edge"].T,
                             preferred_element_type=F32, precision=_PREC)
                     * gp["att_edge"], axis=1, keepdims=True)
             for gp in gl["gat"]], axis=1)
        return sum_ehm + jnp.sum(eh_blk * m_blk, axis=0, keepdims=True)

    sum_ehm = lax.fori_loop(0, E0 // _EB1, edge_blk, jnp.zeros((1, 32), F32))
    mask0_col = m0col_r[...]
    fill = sum_ehm / jnp.sum(mask0_col)

    nh = jnp.tanh(_lin(gl["node_fc1"], shrlat_r[...]))
    nh = jnp.tanh(_lin(gl["node_fc2"], nh))
    h = nh
    hs = [nh]
    for li, gp in enumerate(gl["gat"]):
        ae2d = jnp.reshape(ehm_s[:, 32 + li:33 + li], (C0, C0))
        h = _gat_dense(gp, h, ae2d, fill, mask0_2d, C0)
        hs.append(h)
    lat = jnp.tanh(_lin(gl["latent_fc1"], jnp.concatenate(hs, axis=1)))
    nl1 = jnp.tanh(_lin(gl["latent_fc2"], lat))
    score1 = _cluster_dense(P["cluster"][1], nl1, mask0_2d, C0)
    c1_col = _argmax_col(score1 + g1_r[...], C1)
    s1 = (lax.broadcasted_iota(I32, (C0, C1), 1) == c1_col).astype(F32)
    shr_raw1 = _ddot(s1, nl1)
    nrm = jnp.sqrt(jnp.sum(shr_raw1 * shr_raw1, axis=0, keepdims=True))

    c1row = jnp.reshape(c1_col, (1, C0)).astype(BF)

    def body(i, acc):
        off = i * _EB1
        i_row = jnp.reshape(ii_r[pl.ds(off, _EB1)], (1, _EB1))
        j_row = jnp.reshape(jj_r[pl.ds(off, _EB1)], (1, _EB1))
        ohs_t = _onehot_t(i_row, C0)
        ohd_t = _onehot_t(j_row, C0)
        cs = jnp.dot(c1row, ohs_t, preferred_element_type=F32)
        cd = jnp.dot(c1row, ohd_t, preferred_element_type=F32)
        b_row = (cs * C1 + cd).astype(I32)
        ohb_t = _onehot_t(b_row, C1 * C1)
        payload = ehm_s[pl.ds(off, _EB1), 0:32]
        return acc + _dot_x(ohb_t, payload)

    acc = lax.fori_loop(0, E0 // _EB1, body, jnp.zeros((C1 * C1, 32), F32))
    adj1 = acc / jnp.sum(acc)
    mask1 = (jnp.sum(adj1, axis=1, keepdims=True) != 0).astype(F32)
    sheh1_o[...] = adj1 * mask1
    mask1_o[...] = mask1
    shrlat1_o[...] = shr_raw1 / jnp.maximum(nrm, 1e-12)
    nl1_o[...] = nl1


def _rest_b_body(ins, outs, scr):
    shrlat1_r, sheh1_r, m12d_r, m1col_r, lat0_r, nl1_r, P_r = ins
    pred_o, outlat_o, nl2_o = outs
    P = jax.tree.map(lambda r: r[...], P_r)
    nl2, _ = _encoder_dense(P["glob"][1], shrlat1_r[...], sheh1_r[...],
                            m12d_r[...], m1col_r[...], C1)
    latents = jnp.concatenate(
        [lat0_r[...], jnp.sum(nl1_r[...], axis=0, keepdims=True),
         jnp.sum(nl2, axis=0, keepdims=True)], axis=0)
    mp = P["mha"]
    d = 64
    w, bb = mp["in_w"], mp["in_b"]
    q = jnp.dot(latents, w[:d].T, preferred_element_type=F32, precision=_PREC) + bb[:d]
    k = jnp.dot(latents, w[d:2 * d].T, preferred_element_type=F32, precision=_PREC) + bb[d:2 * d]
    v = jnp.dot(latents, w[2 * d:].T, preferred_element_type=F32, precision=_PREC) + bb[2 * d:]
    logits = jnp.dot(q, k.T, preferred_element_type=F32, precision=_PREC) / jnp.sqrt(F32(d))
    logits = logits - jnp.max(logits, axis=1, keepdims=True)
    el = jnp.exp(logits)
    a = el / jnp.sum(el, axis=1, keepdims=True)
    att = (jnp.dot(jnp.dot(a, v, preferred_element_type=F32, precision=_PREC), mp["out_w"].T,
                   preferred_element_type=F32, precision=_PREC) + mp["out_b"])
    out_lat = jnp.mean(att, axis=0, keepdims=True)
    hid = jnp.tanh(_lin(P["fc1"], out_lat))
    pred_o[...] = _lin(P["fc2"], hid)
    outlat_o[...] = out_lat
    nl2_o[...] = nl2


def kernel(adj, node_feat, edge_feat, params):
    adj = adj.astype(I32)
    src1d, dst1d = adj[0], adj[1]
    u0 = jax.random.uniform(jax.random.fold_in(jax.random.key(7), 0),
                            (N0, C0), minval=1e-10, maxval=1.0)
    g0 = -jnp.log(-jnp.log(u0))
    u1 = jax.random.uniform(jax.random.fold_in(jax.random.key(7), 1),
                            (C0, C1), minval=1e-10, maxval=1.0)
    g1 = -jnp.log(-jnp.log(u1))
    ii1d = jnp.repeat(jnp.arange(C0, dtype=I32), C0)
    jj1d = jnp.tile(jnp.arange(C0, dtype=I32), C0)

    base_lat, shr_lat, c0_1d, eh, lat0_sum = _run(
        _level0_body,
        [((N0, 64), F32), ((C0, 64), F32), ((N0,), I32),
         ((E0, 32), F32), ((1, 64), F32)],
        [pltpu.VMEM((E0, 2), F32)],
        src1d, dst1d, node_feat, edge_feat, g0, params)

    adj2, mask0_2d = _run(
        _pool0_body,
        [((C0, C0 * 32), F32), ((C0, C0), F32)],
        [],
        src1d, dst1d, c0_1d, eh)
    adj0_flat = adj2.reshape(C0, 32, C0).transpose(0, 2, 1).reshape(C0 * C0, 32)
    mask0_col = mask0_2d.reshape(C0 * C0, 1)

    shr_eh1, mask1, shr_lat1, nl1 = _run(
        _rest_a_body,
        [((C1 * C1, 32), F32), ((C1 * C1, 1), F32), ((C1, 64), F32),
         ((C0, 64), F32)],
        [pltpu.VMEM((E0, 34), F32)],
        shr_lat, adj0_flat, mask0_2d, mask0_col, ii1d, jj1d, g1,
        params)

    pred, out_lat, nl2 = _run(
        _rest_b_body,
        [((1, 16), F32), ((1, 64), F32), ((C1, 64), F32)],
        [],
        shr_lat1, shr_eh1, mask1.reshape(C1, C1), mask1, lat0_sum, nl1,
        params)

    return (pred.reshape(16), out_lat.reshape(64), base_lat, nl1, nl2)
